# baseline (speedup 1.0000x reference)
import functools

import jax
import jax.numpy as jnp
from jax import lax
from jax.experimental import pallas as pl
from jax.experimental.pallas import tpu as pltpu
from jax.experimental.pallas import tpu_sc as plsc

N = 10000
E = 320000
D = 128

NC = 2
NS = 16
NW = NC * NS
NP = 10240
EP = NW * NP
EPAD = EP - E
C = 128
G = 8
TOTCH = EP // C
NCH0 = 160
NG0 = NCH0 // G
RPT = NP // NS
ZR = 8
DR = NP // D
L = 16


def _agg_body(compute_deg, *refs):
    if compute_deg:
        (feat, srcI, dstI, out, degout,
         srcg0, dstg0, srcg1, dstg1, rowsA, rowsB, zbuf, degl, acc,
         semA, semB, semI0, semI1) = refs
    else:
        (feat, srcI, dstI, out,
         srcg0, dstg0, srcg1, dstg1, rowsA, rowsB, zbuf, acc,
         semA, semB, semI0, semI1) = refs
    c = lax.axis_index("c")
    s = lax.axis_index("s")

    zeros = jnp.zeros((L,), jnp.float32)
    ones = jnp.ones((L,), jnp.float32)

    @pl.when(c == 0)
    def _():
        def zrow(i, _):
            def zcol(j, _):
                zbuf[i, pl.ds(j * L, L)] = zeros
                return 0
            return lax.fori_loop(0, D // L, zcol, 0)

        lax.fori_loop(0, ZR, zrow, 0)

        def zslice(r, _):
            pltpu.sync_copy(zbuf, acc.at[pl.ds(s * RPT + r * ZR, ZR)])
            return 0

        lax.fori_loop(0, RPT // ZR, zslice, 0)

        if compute_deg:
            def zdeg(i, _):
                def zdcol(j, _):
                    degl[i, pl.ds(j * L, L)] = zeros
                    return 0
                return lax.fori_loop(0, D // L, zdcol, 0)
            lax.fori_loop(0, DR, zdeg, 0)

    plsc.subcore_barrier()

    def fire_gather(sg, row, rbuf, sem):
        pltpu.async_copy(feat.at[sg.at[row]], rbuf, sem)

    def wait_gather(sg, rbuf, sem):
        pltpu.make_async_copy(feat.at[sg.at[0]], rbuf, sem).wait()

    def do_scatter(rbuf, dg, row):
        pltpu.sync_copy(rbuf, acc.at[dg.at[row]], add=True)
        if compute_deg:
            def dv(i, _):
                d = dg[row, pl.ds(i * L, L)]
                plsc.addupdate_scatter(
                    degl,
                    [lax.shift_right_logical(d, 7),
                     lax.bitwise_and(d, 127)],
                    ones)
                return 0
            lax.fori_loop(0, C // L, dv, 0)

    def run_pipeline(cb, ng):
        def fire_stage(g, sg, dg, sem):
            pltpu.async_copy(srcI.at[pl.ds(cb + g * G, G)], sg, sem)
            pltpu.async_copy(dstI.at[pl.ds(cb + g * G, G)], dg, sem)

        def wait_stage(sg, dg, sem):
            pltpu.make_async_copy(srcI.at[pl.ds(cb, G)], sg, sem).wait()
            pltpu.make_async_copy(dstI.at[pl.ds(cb, G)], dg, sem).wait()

        def block(sg, dg, nsg, ndg, semNxt, fire_next, restage_g,
                  restage_sem):
            def pair(kk, _):
                fire_gather(sg, 2 * kk + 1, rowsB, semB)
                wait_gather(sg, rowsA, semA)
                do_scatter(rowsA, dg, 2 * kk)
                fire_gather(sg, 2 * kk + 2, rowsA, semA)
                wait_gather(sg, rowsB, semB)
                do_scatter(rowsB, dg, 2 * kk + 1)
                return 0

            lax.fori_loop(0, G // 2 - 1, pair, 0)
            fire_gather(sg, G - 1, rowsB, semB)
            wait_gather(sg, rowsA, semA)
            do_scatter(rowsA, dg, G - 2)

            @pl.when(fire_next)
            def _():
                wait_stage(nsg, ndg, semNxt)
                fire_gather(nsg, 0, rowsA, semA)

            wait_gather(sg, rowsB, semB)
            do_scatter(rowsB, dg, G - 1)

            @pl.when(restage_g < ng)
            def _():
                fire_stage(restage_g, sg, dg, restage_sem)

        pltpu.sync_copy(srcI.at[pl.ds(cb, G)], srcg0)
        pltpu.sync_copy(dstI.at[pl.ds(cb, G)], dstg0)
        fire_stage(1, srcg1, dstg1, semI1)
        fire_gather(srcg0, 0, rowsA, semA)

        ng2 = ng // 2

        def gloop(gp, _):
            block(srcg0, dstg0, srcg1, dstg1, semI1,
                  fire_next=gp < ng2, restage_g=2 * gp + 2,
                  restage_sem=semI0)
            block(srcg1, dstg1, srcg0, dstg0, semI0,
                  fire_next=gp < ng2 - 1, restage_g=2 * gp + 3,
                  restage_sem=semI1)
            return 0

        lax.fori_loop(0, ng2, gloop, 0)

    @pl.when(c == 0)
    def _():
        run_pipeline(s * NCH0, NG0)

    plsc.subcore_barrier()

    @pl.when(c == 0)
    def _():
        pltpu.sync_copy(acc.at[pl.ds(s * RPT, RPT)],
                        out.at[pl.ds(s * RPT, RPT)])
        if compute_deg:
            pltpu.sync_copy(degl, degout.at[s])


@functools.lru_cache(maxsize=None)
def _make_agg(compute_deg):
    mesh = plsc.VectorSubcoreMesh(core_axis_name="c", subcore_axis_name="s",
                                  num_cores=NC, num_subcores=NS)
    out_type = [jax.ShapeDtypeStruct((NP, D), jnp.float32)]
    scratch = [
        pltpu.VMEM((G, C), jnp.int32),
        pltpu.VMEM((G, C), jnp.int32),
        pltpu.VMEM((G, C), jnp.int32),
        pltpu.VMEM((G, C), jnp.int32),
        pltpu.VMEM((C, D), jnp.float32),
        pltpu.VMEM((C, D), jnp.float32),
        pltpu.VMEM((ZR, D), jnp.float32),
    ]
    if compute_deg:
        out_type.append(jax.ShapeDtypeStruct((NS, DR, D), jnp.float32))
        scratch.append(pltpu.VMEM((DR, D), jnp.float32))
    scratch.append(pltpu.VMEM_SHARED((NP, D), jnp.float32))
    scratch.extend([pltpu.SemaphoreType.DMA] * 4)
    return pl.kernel(
        functools.partial(_agg_body, compute_deg),
        out_type=tuple(out_type),
        mesh=mesh,
        scratch_types=tuple(scratch),
        compiler_params=pltpu.CompilerParams(needs_layout_passes=False,
                                             use_tc_tiling_on_sc=False),
    )


def _tc1_body(p, inv, x, wl, bl, wr, out):
    agg = p[...] * inv[...]
    h = (jnp.dot(agg, wl[...], preferred_element_type=jnp.float32)
         + bl[...]
         + jnp.dot(x[...], wr[...], preferred_element_type=jnp.float32))
    out[...] = jnp.maximum(h, 0.0)


def _tc2_body(q, inv, e1, wl2, bl2, wr2, wfc1, bfc1, g1, be1,
              wfc2, bfc2, g2, be2, wfc3, bfc3, emb2, out):
    agg = q[...] * inv[...]
    h = (jnp.dot(agg, wl2[...], preferred_element_type=jnp.float32)
         + bl2[...]
         + jnp.dot(e1[...], wr2[...], preferred_element_type=jnp.float32))
    h = jnp.maximum(h, 0.0)
    emb2[...] = h

    y = jnp.dot(h, wfc1[...], preferred_element_type=jnp.float32) + bfc1[...]
    mu = jnp.mean(y, axis=0, keepdims=True)
    var = jnp.mean((y - mu) ** 2, axis=0, keepdims=True)
    y = g1[...] * (y - mu) / jnp.sqrt(var + 1e-5) + be1[...]
    y = jnp.maximum(y, 0.0)

    y = jnp.dot(y, wfc2[...], preferred_element_type=jnp.float32) + bfc2[...]
    mu = jnp.mean(y, axis=0, keepdims=True)
    var = jnp.mean((y - mu) ** 2, axis=0, keepdims=True)
    y = g2[...] * (y - mu) / jnp.sqrt(var + 1e-5) + be2[...]
    y = jnp.maximum(y, 0.0)

    out[...] = jnp.dot(y, wfc3[...], preferred_element_type=jnp.float32) + bfc3[...]


def kernel(x, edge_index, W_l1, b_l1, W_r1, W_l2, b_l2, W_r2,
           W_fc1, b_fc1, g_bn1, be_bn1, W_fc2, b_fc2, g_bn2, be_bn2,
           W_fc3, b_fc3):
    ei = edge_index.astype(jnp.int32)
    srcI = jnp.concatenate(
        [ei[0], jnp.zeros((EPAD,), jnp.int32)]).reshape(TOTCH, C)
    dstI = jnp.concatenate(
        [ei[1], jnp.full((EPAD,), N, jnp.int32)]).reshape(TOTCH, C)

    p1, degp = _make_agg(True)(x, srcI, dstI)

    deg = degp.sum(axis=0).reshape(NP)[:N]
    inv = (1.0 / jnp.maximum(deg, 1.0)).reshape(N, 1)

    def _full(a):
        return pl.BlockSpec(a.shape, lambda i: (0,) * a.ndim)

    _pspec = pl.BlockSpec((N, D), lambda i: (0, 0))

    tc1_in = [p1, inv, x, W_l1, b_l1.reshape(1, -1), W_r1]
    emb1 = pl.pallas_call(
        _tc1_body,
        out_shape=jax.ShapeDtypeStruct((N, D), jnp.float32),
        grid=(1,),
        in_specs=[_pspec] + [_full(a) for a in tc1_in[1:]],
        out_specs=pl.BlockSpec((N, D), lambda i: (0, 0)),
    )(*tc1_in)

    (q,) = _make_agg(False)(emb1, srcI, dstI)

    tc2_in = [q, inv, emb1, W_l2, b_l2.reshape(1, -1), W_r2,
              W_fc1, b_fc1.reshape(1, -1), g_bn1.reshape(1, -1),
              be_bn1.reshape(1, -1), W_fc2, b_fc2.reshape(1, -1),
              g_bn2.reshape(1, -1), be_bn2.reshape(1, -1),
              W_fc3, b_fc3.reshape(1, -1)]
    emb2, out = pl.pallas_call(
        _tc2_body,
        out_shape=(
            jax.ShapeDtypeStruct((N, D), jnp.float32),
            jax.ShapeDtypeStruct((N, 1), jnp.float32),
        ),
        grid=(1,),
        in_specs=[_pspec] + [_full(a) for a in tc2_in[1:]],
        out_specs=(pl.BlockSpec((N, D), lambda i: (0, 0)),
                   pl.BlockSpec((N, 1), lambda i: (0, 0))),
    )(*tc2_in)

    return (out.reshape(N), emb1, emb2)


# speedup vs baseline: 1.1331x; 1.1331x over previous
import functools

import jax
import jax.numpy as jnp
from jax import lax
from jax.experimental import pallas as pl
from jax.experimental.pallas import tpu as pltpu
from jax.experimental.pallas import tpu_sc as plsc

N = 10000
E = 320000
D = 128

NC = 2
NS = 16
NW = NC * NS
NP = 10240
EP = NW * NP
EPAD = EP - E
C = 128
G = 8
TOTCH = EP // C
NCH0 = 112
NCH1 = 48
NG0 = NCH0 // G
NG1 = NCH1 // G
RPT = NP // NS
ZR = 8
DR = NP // D
L = 16


def _agg_body(compute_deg, *refs):
    if compute_deg:
        (feat, srcI, dstI, out, degout,
         srcg0, dstg0, srcg1, dstg1, rowsA, rowsB, zbuf, degl, acc,
         semA, semB, semI0, semI1) = refs
    else:
        (feat, srcI, dstI, out,
         srcg0, dstg0, srcg1, dstg1, rowsA, rowsB, zbuf, acc,
         semA, semB, semI0, semI1) = refs
    c = lax.axis_index("c")
    s = lax.axis_index("s")

    zeros = jnp.zeros((L,), jnp.float32)
    ones = jnp.ones((L,), jnp.float32)

    if True:
        def zrow(i, _):
            def zcol(j, _):
                zbuf[i, pl.ds(j * L, L)] = zeros
                return 0
            return lax.fori_loop(0, D // L, zcol, 0)

        lax.fori_loop(0, ZR, zrow, 0)

        def zslice(r, _):
            pltpu.sync_copy(zbuf, acc.at[pl.ds(s * RPT + r * ZR, ZR)])
            return 0

        lax.fori_loop(0, RPT // ZR, zslice, 0)

        if compute_deg:
            def zdeg(i, _):
                def zdcol(j, _):
                    degl[i, pl.ds(j * L, L)] = zeros
                    return 0
                return lax.fori_loop(0, D // L, zdcol, 0)
            lax.fori_loop(0, DR, zdeg, 0)

    plsc.subcore_barrier()

    def fire_gather(sg, row, rbuf, sem):
        pltpu.async_copy(feat.at[sg.at[row]], rbuf, sem)

    def wait_gather(sg, rbuf, sem):
        pltpu.make_async_copy(feat.at[sg.at[0]], rbuf, sem).wait()

    def do_scatter(rbuf, dg, row):
        pltpu.sync_copy(rbuf, acc.at[dg.at[row]], add=True)
        if compute_deg:
            def dv(i, _):
                d = dg[row, pl.ds(i * L, L)]
                plsc.addupdate_scatter(
                    degl,
                    [lax.shift_right_logical(d, 7),
                     lax.bitwise_and(d, 127)],
                    ones)
                return 0
            lax.fori_loop(0, C // L, dv, 0)

    def run_pipeline(cb, ng):
        def fire_stage(g, sg, dg, sem):
            pltpu.async_copy(srcI.at[pl.ds(cb + g * G, G)], sg, sem)
            pltpu.async_copy(dstI.at[pl.ds(cb + g * G, G)], dg, sem)

        def wait_stage(sg, dg, sem):
            pltpu.make_async_copy(srcI.at[pl.ds(cb, G)], sg, sem).wait()
            pltpu.make_async_copy(dstI.at[pl.ds(cb, G)], dg, sem).wait()

        def block(sg, dg, nsg, ndg, semNxt, fire_next, restage_g,
                  restage_sem):
            def pair(kk, _):
                fire_gather(sg, 2 * kk + 1, rowsB, semB)
                wait_gather(sg, rowsA, semA)
                do_scatter(rowsA, dg, 2 * kk)
                fire_gather(sg, 2 * kk + 2, rowsA, semA)
                wait_gather(sg, rowsB, semB)
                do_scatter(rowsB, dg, 2 * kk + 1)
                return 0

            lax.fori_loop(0, G // 2 - 1, pair, 0)
            fire_gather(sg, G - 1, rowsB, semB)
            wait_gather(sg, rowsA, semA)
            do_scatter(rowsA, dg, G - 2)

            @pl.when(fire_next)
            def _():
                wait_stage(nsg, ndg, semNxt)
                fire_gather(nsg, 0, rowsA, semA)

            wait_gather(sg, rowsB, semB)
            do_scatter(rowsB, dg, G - 1)

            @pl.when(restage_g < ng)
            def _():
                fire_stage(restage_g, sg, dg, restage_sem)

        pltpu.sync_copy(srcI.at[pl.ds(cb, G)], srcg0)
        pltpu.sync_copy(dstI.at[pl.ds(cb, G)], dstg0)
        fire_stage(1, srcg1, dstg1, semI1)
        fire_gather(srcg0, 0, rowsA, semA)

        ng2 = ng // 2

        def gloop(gp, _):
            block(srcg0, dstg0, srcg1, dstg1, semI1,
                  fire_next=gp < ng2, restage_g=2 * gp + 2,
                  restage_sem=semI0)
            block(srcg1, dstg1, srcg0, dstg0, semI0,
                  fire_next=gp < ng2 - 1, restage_g=2 * gp + 3,
                  restage_sem=semI1)
            return 0

        lax.fori_loop(0, ng2, gloop, 0)

    @pl.when(c == 0)
    def _():
        run_pipeline(s * NCH0, NG0)

    @pl.when(c == 1)
    def _():
        run_pipeline(NS * NCH0 + s * NCH1, NG1)

    plsc.subcore_barrier()

    pltpu.sync_copy(acc.at[pl.ds(s * RPT, RPT)],
                    out.at[c, pl.ds(s * RPT, RPT)])
    if compute_deg:
        pltpu.sync_copy(degl, degout.at[s * NC + c])


@functools.lru_cache(maxsize=None)
def _make_agg(compute_deg):
    mesh = plsc.VectorSubcoreMesh(core_axis_name="c", subcore_axis_name="s",
                                  num_cores=NC, num_subcores=NS)
    out_type = [jax.ShapeDtypeStruct((NC, NP, D), jnp.float32)]
    scratch = [
        pltpu.VMEM((G, C), jnp.int32),
        pltpu.VMEM((G, C), jnp.int32),
        pltpu.VMEM((G, C), jnp.int32),
        pltpu.VMEM((G, C), jnp.int32),
        pltpu.VMEM((C, D), jnp.float32),
        pltpu.VMEM((C, D), jnp.float32),
        pltpu.VMEM((ZR, D), jnp.float32),
    ]
    if compute_deg:
        out_type.append(jax.ShapeDtypeStruct((NW, DR, D), jnp.float32))
        scratch.append(pltpu.VMEM((DR, D), jnp.float32))
    scratch.append(pltpu.VMEM_SHARED((NP, D), jnp.float32))
    scratch.extend([pltpu.SemaphoreType.DMA] * 4)
    return pl.kernel(
        functools.partial(_agg_body, compute_deg),
        out_type=tuple(out_type),
        mesh=mesh,
        scratch_types=tuple(scratch),
        compiler_params=pltpu.CompilerParams(needs_layout_passes=False,
                                             use_tc_tiling_on_sc=False),
    )


def _tc1_body(p, inv, x, wl, bl, wr, out):
    agg = (p[0] + p[1]) * inv[...]
    h = (jnp.dot(agg, wl[...], preferred_element_type=jnp.float32)
         + bl[...]
         + jnp.dot(x[...], wr[...], preferred_element_type=jnp.float32))
    out[...] = jnp.maximum(h, 0.0)


def _tc2_body(q, inv, e1, wl2, bl2, wr2, wfc1, bfc1, g1, be1,
              wfc2, bfc2, g2, be2, wfc3, bfc3, emb2, out):
    agg = (q[0] + q[1]) * inv[...]
    h = (jnp.dot(agg, wl2[...], preferred_element_type=jnp.float32)
         + bl2[...]
         + jnp.dot(e1[...], wr2[...], preferred_element_type=jnp.float32))
    h = jnp.maximum(h, 0.0)
    emb2[...] = h

    y = jnp.dot(h, wfc1[...], preferred_element_type=jnp.float32) + bfc1[...]
    mu = jnp.mean(y, axis=0, keepdims=True)
    var = jnp.mean((y - mu) ** 2, axis=0, keepdims=True)
    y = g1[...] * (y - mu) / jnp.sqrt(var + 1e-5) + be1[...]
    y = jnp.maximum(y, 0.0)

    y = jnp.dot(y, wfc2[...], preferred_element_type=jnp.float32) + bfc2[...]
    mu = jnp.mean(y, axis=0, keepdims=True)
    var = jnp.mean((y - mu) ** 2, axis=0, keepdims=True)
    y = g2[...] * (y - mu) / jnp.sqrt(var + 1e-5) + be2[...]
    y = jnp.maximum(y, 0.0)

    out[...] = jnp.dot(y, wfc3[...], preferred_element_type=jnp.float32) + bfc3[...]


def kernel(x, edge_index, W_l1, b_l1, W_r1, W_l2, b_l2, W_r2,
           W_fc1, b_fc1, g_bn1, be_bn1, W_fc2, b_fc2, g_bn2, be_bn2,
           W_fc3, b_fc3):
    ei = edge_index.astype(jnp.int32)
    srcI = jnp.concatenate(
        [ei[0], jnp.zeros((EPAD,), jnp.int32)]).reshape(TOTCH, C)
    dstI = jnp.concatenate(
        [ei[1], jnp.full((EPAD,), N, jnp.int32)]).reshape(TOTCH, C)

    p1, degp = _make_agg(True)(x, srcI, dstI)

    deg = degp.sum(axis=0).reshape(NP)[:N]
    inv = (1.0 / jnp.maximum(deg, 1.0)).reshape(N, 1)

    def _full(a):
        return pl.BlockSpec(a.shape, lambda i: (0,) * a.ndim)

    _pspec = pl.BlockSpec((NC, N, D), lambda i: (0, 0, 0))

    tc1_in = [p1, inv, x, W_l1, b_l1.reshape(1, -1), W_r1]
    emb1 = pl.pallas_call(
        _tc1_body,
        out_shape=jax.ShapeDtypeStruct((N, D), jnp.float32),
        grid=(1,),
        in_specs=[_pspec] + [_full(a) for a in tc1_in[1:]],
        out_specs=pl.BlockSpec((N, D), lambda i: (0, 0)),
    )(*tc1_in)

    (q,) = _make_agg(False)(emb1, srcI, dstI)

    tc2_in = [q, inv, emb1, W_l2, b_l2.reshape(1, -1), W_r2,
              W_fc1, b_fc1.reshape(1, -1), g_bn1.reshape(1, -1),
              be_bn1.reshape(1, -1), W_fc2, b_fc2.reshape(1, -1),
              g_bn2.reshape(1, -1), be_bn2.reshape(1, -1),
              W_fc3, b_fc3.reshape(1, -1)]
    emb2, out = pl.pallas_call(
        _tc2_body,
        out_shape=(
            jax.ShapeDtypeStruct((N, D), jnp.float32),
            jax.ShapeDtypeStruct((N, 1), jnp.float32),
        ),
        grid=(1,),
        in_specs=[_pspec] + [_full(a) for a in tc2_in[1:]],
        out_specs=(pl.BlockSpec((N, D), lambda i: (0, 0)),
                   pl.BlockSpec((N, 1), lambda i: (0, 0))),
    )(*tc2_in)

    return (out.reshape(N), emb1, emb2)


# speedup vs baseline: 1.3261x; 1.1704x over previous
import functools

import numpy as np

import jax
import jax.numpy as jnp
from jax import lax
from jax.experimental import pallas as pl
from jax.experimental.pallas import tpu as pltpu
from jax.experimental.pallas import tpu_sc as plsc

N = 10000
E = 320000
D = 128

NC = 2
NS = 16
NW = NC * NS
NP = 10240
EP = NW * NP
EPAD = EP - E
C = 128
G = 8
TOTCH = EP // C
NCH0 = 112
NCH1 = 48
NG0 = NCH0 // G
NG1 = NCH1 // G
RPT = NP // NS
ZR = 8
DR = NP // D
L = 16

_M = np.empty((D,), np.int64)
for _j in range(D // 32):
    for _r in range(32):
        _M[32 * _j + _r] = 32 * _j + (2 * _r if _r < 16 else 2 * (_r - 16) + 1)
_MINV = np.argsort(_M)
_PM = np.zeros((D, D), np.float32)
for _b in range(D):
    _PM[_MINV[_b], _b] = 1.0


def _agg_body(compute_deg, *refs):
    if compute_deg:
        (feat, srcI, dstI, out, degout,
         srcg0, dstg0, srcg1, dstg1, rowsA, rowsB, rowsF, zbuf, degl, acc,
         semA, semB, semI0, semI1) = refs
    else:
        (feat, srcI, dstI, out,
         srcg0, dstg0, srcg1, dstg1, rowsA, rowsB, rowsF, zbuf, acc,
         semA, semB, semI0, semI1) = refs
    c = lax.axis_index("c")
    s = lax.axis_index("s")

    zeros = jnp.zeros((L,), jnp.float32)
    ones = jnp.ones((L,), jnp.float32)

    if True:
        def zrow(i, _):
            def zcol(j, _):
                zbuf[i, pl.ds(j * L, L)] = zeros
                return 0
            return lax.fori_loop(0, D // L, zcol, 0)

        lax.fori_loop(0, ZR, zrow, 0)

        def zslice(r, _):
            pltpu.sync_copy(zbuf, acc.at[pl.ds(s * RPT + r * ZR, ZR)])
            return 0

        lax.fori_loop(0, RPT // ZR, zslice, 0)

        if compute_deg:
            def zdeg(i, _):
                def zdcol(j, _):
                    degl[i, pl.ds(j * L, L)] = zeros
                    return 0
                return lax.fori_loop(0, D // L, zdcol, 0)
            lax.fori_loop(0, DR, zdeg, 0)

    plsc.subcore_barrier()

    def fire_gather(sg, row, rbuf, sem):
        pltpu.async_copy(feat.at[sg.at[row]], rbuf, sem)

    def wait_gather(sg, rbuf, sem):
        pltpu.make_async_copy(feat.at[sg.at[0]], rbuf, sem).wait()

    def do_scatter(rbuf, dg, row):
        mask = jnp.full((L,), -65536, jnp.int32)
        def conv_row(r, _):
            for j in range(D // 32):
                b = rbuf[r, pl.ds(j * 32, 32)]
                w = plsc.bitcast(b, jnp.int32)
                lo = lax.shift_left(w, 16)
                hi = lax.bitwise_and(w, mask)
                rowsF[r, pl.ds(j * 32, L)] = plsc.bitcast(lo, jnp.float32)
                rowsF[r, pl.ds(j * 32 + L, L)] = plsc.bitcast(hi, jnp.float32)
            return 0
        lax.fori_loop(0, C, conv_row, 0)
        pltpu.sync_copy(rowsF, acc.at[dg.at[row]], add=True)
        if compute_deg:
            def dv(i, _):
                d = dg[row, pl.ds(i * L, L)]
                plsc.addupdate_scatter(
                    degl,
                    [lax.shift_right_logical(d, 7),
                     lax.bitwise_and(d, 127)],
                    ones)
                return 0
            lax.fori_loop(0, C // L, dv, 0)

    def run_pipeline(cb, ng):
        def fire_stage(g, sg, dg, sem):
            pltpu.async_copy(srcI.at[pl.ds(cb + g * G, G)], sg, sem)
            pltpu.async_copy(dstI.at[pl.ds(cb + g * G, G)], dg, sem)

        def wait_stage(sg, dg, sem):
            pltpu.make_async_copy(srcI.at[pl.ds(cb, G)], sg, sem).wait()
            pltpu.make_async_copy(dstI.at[pl.ds(cb, G)], dg, sem).wait()

        def block(sg, dg, nsg, ndg, semNxt, fire_next, restage_g,
                  restage_sem):
            def pair(kk, _):
                fire_gather(sg, 2 * kk + 1, rowsB, semB)
                wait_gather(sg, rowsA, semA)
                do_scatter(rowsA, dg, 2 * kk)
                fire_gather(sg, 2 * kk + 2, rowsA, semA)
                wait_gather(sg, rowsB, semB)
                do_scatter(rowsB, dg, 2 * kk + 1)
                return 0

            lax.fori_loop(0, G // 2 - 1, pair, 0)
            fire_gather(sg, G - 1, rowsB, semB)
            wait_gather(sg, rowsA, semA)
            do_scatter(rowsA, dg, G - 2)

            @pl.when(fire_next)
            def _():
                wait_stage(nsg, ndg, semNxt)
                fire_gather(nsg, 0, rowsA, semA)

            wait_gather(sg, rowsB, semB)
            do_scatter(rowsB, dg, G - 1)

            @pl.when(restage_g < ng)
            def _():
                fire_stage(restage_g, sg, dg, restage_sem)

        pltpu.sync_copy(srcI.at[pl.ds(cb, G)], srcg0)
        pltpu.sync_copy(dstI.at[pl.ds(cb, G)], dstg0)
        fire_stage(1, srcg1, dstg1, semI1)
        fire_gather(srcg0, 0, rowsA, semA)

        ng2 = ng // 2

        def gloop(gp, _):
            block(srcg0, dstg0, srcg1, dstg1, semI1,
                  fire_next=gp < ng2, restage_g=2 * gp + 2,
                  restage_sem=semI0)
            block(srcg1, dstg1, srcg0, dstg0, semI0,
                  fire_next=gp < ng2 - 1, restage_g=2 * gp + 3,
                  restage_sem=semI1)
            return 0

        lax.fori_loop(0, ng2, gloop, 0)

    @pl.when(c == 0)
    def _():
        run_pipeline(s * NCH0, NG0)

    @pl.when(c == 1)
    def _():
        run_pipeline(NS * NCH0 + s * NCH1, NG1)

    plsc.subcore_barrier()

    pltpu.sync_copy(acc.at[pl.ds(s * RPT, RPT)],
                    out.at[c, pl.ds(s * RPT, RPT)])
    if compute_deg:
        pltpu.sync_copy(degl, degout.at[s * NC + c])


@functools.lru_cache(maxsize=None)
def _make_agg(compute_deg):
    mesh = plsc.VectorSubcoreMesh(core_axis_name="c", subcore_axis_name="s",
                                  num_cores=NC, num_subcores=NS)
    out_type = [jax.ShapeDtypeStruct((NC, NP, D), jnp.float32)]
    scratch = [
        pltpu.VMEM((G, C), jnp.int32),
        pltpu.VMEM((G, C), jnp.int32),
        pltpu.VMEM((G, C), jnp.int32),
        pltpu.VMEM((G, C), jnp.int32),
        pltpu.VMEM((C, D), jnp.bfloat16),
        pltpu.VMEM((C, D), jnp.bfloat16),
        pltpu.VMEM((C, D), jnp.float32),
        pltpu.VMEM((ZR, D), jnp.float32),
    ]
    if compute_deg:
        out_type.append(jax.ShapeDtypeStruct((NW, DR, D), jnp.float32))
        scratch.append(pltpu.VMEM((DR, D), jnp.float32))
    scratch.append(pltpu.VMEM_SHARED((NP, D), jnp.float32))
    scratch.extend([pltpu.SemaphoreType.DMA] * 4)
    return pl.kernel(
        functools.partial(_agg_body, compute_deg),
        out_type=tuple(out_type),
        mesh=mesh,
        scratch_types=tuple(scratch),
        compiler_params=pltpu.CompilerParams(needs_layout_passes=False,
                                             use_tc_tiling_on_sc=False),
    )


def _tc1_body(p, inv, x, wl, bl, wr, pm, out, outp):
    agg = (p[0] + p[1]) * inv[...]
    h = (jnp.dot(agg, wl[...], preferred_element_type=jnp.float32)
         + bl[...]
         + jnp.dot(x[...], wr[...], preferred_element_type=jnp.float32))
    h = jnp.maximum(h, 0.0)
    out[...] = h
    outp[...] = jnp.dot(h, pm[...],
                        preferred_element_type=jnp.float32).astype(jnp.bfloat16)


def _tc2_body(q, inv, e1, wl2, bl2, wr2, wfc1, bfc1, g1, be1,
              wfc2, bfc2, g2, be2, wfc3, bfc3, emb2, out):
    agg = (q[0] + q[1]) * inv[...]
    h = (jnp.dot(agg, wl2[...], preferred_element_type=jnp.float32)
         + bl2[...]
         + jnp.dot(e1[...], wr2[...], preferred_element_type=jnp.float32))
    h = jnp.maximum(h, 0.0)
    emb2[...] = h

    y = jnp.dot(h, wfc1[...], preferred_element_type=jnp.float32) + bfc1[...]
    mu = jnp.mean(y, axis=0, keepdims=True)
    var = jnp.mean((y - mu) ** 2, axis=0, keepdims=True)
    y = g1[...] * (y - mu) / jnp.sqrt(var + 1e-5) + be1[...]
    y = jnp.maximum(y, 0.0)

    y = jnp.dot(y, wfc2[...], preferred_element_type=jnp.float32) + bfc2[...]
    mu = jnp.mean(y, axis=0, keepdims=True)
    var = jnp.mean((y - mu) ** 2, axis=0, keepdims=True)
    y = g2[...] * (y - mu) / jnp.sqrt(var + 1e-5) + be2[...]
    y = jnp.maximum(y, 0.0)

    out[...] = jnp.dot(y, wfc3[...], preferred_element_type=jnp.float32) + bfc3[...]


def kernel(x, edge_index, W_l1, b_l1, W_r1, W_l2, b_l2, W_r2,
           W_fc1, b_fc1, g_bn1, be_bn1, W_fc2, b_fc2, g_bn2, be_bn2,
           W_fc3, b_fc3):
    ei = edge_index.astype(jnp.int32)
    srcI = jnp.concatenate(
        [ei[0], jnp.zeros((EPAD,), jnp.int32)]).reshape(TOTCH, C)
    dstI = jnp.concatenate(
        [ei[1], jnp.full((EPAD,), N, jnp.int32)]).reshape(TOTCH, C)

    xp_bf = x[:, _MINV].astype(jnp.bfloat16)
    p1, degp = _make_agg(True)(xp_bf, srcI, dstI)

    deg = degp.sum(axis=0).reshape(NP)[:N]
    inv = (1.0 / jnp.maximum(deg, 1.0)).reshape(N, 1)

    def _full(a):
        return pl.BlockSpec(a.shape, lambda i: (0,) * a.ndim)

    _pspec = pl.BlockSpec((NC, N, D), lambda i: (0, 0, 0))

    tc1_in = [p1, inv, x, W_l1, b_l1.reshape(1, -1), W_r1, jnp.asarray(_PM)]
    emb1, emb1p_bf = pl.pallas_call(
        _tc1_body,
        out_shape=(jax.ShapeDtypeStruct((N, D), jnp.float32),
                   jax.ShapeDtypeStruct((N, D), jnp.bfloat16)),
        grid=(1,),
        in_specs=[_pspec] + [_full(a) for a in tc1_in[1:]],
        out_specs=(pl.BlockSpec((N, D), lambda i: (0, 0)),
                   pl.BlockSpec((N, D), lambda i: (0, 0))),
    )(*tc1_in)

    (q,) = _make_agg(False)(emb1p_bf, srcI, dstI)

    tc2_in = [q, inv, emb1, W_l2, b_l2.reshape(1, -1), W_r2,
              W_fc1, b_fc1.reshape(1, -1), g_bn1.reshape(1, -1),
              be_bn1.reshape(1, -1), W_fc2, b_fc2.reshape(1, -1),
              g_bn2.reshape(1, -1), be_bn2.reshape(1, -1),
              W_fc3, b_fc3.reshape(1, -1)]
    emb2, out = pl.pallas_call(
        _tc2_body,
        out_shape=(
            jax.ShapeDtypeStruct((N, D), jnp.float32),
            jax.ShapeDtypeStruct((N, 1), jnp.float32),
        ),
        grid=(1,),
        in_specs=[_pspec] + [_full(a) for a in tc2_in[1:]],
        out_specs=(pl.BlockSpec((N, D), lambda i: (0, 0)),
                   pl.BlockSpec((N, 1), lambda i: (0, 0))),
    )(*tc2_in)

    return (out.reshape(N), emb1, emb2)


# speedup vs baseline: 1.9055x; 1.4369x over previous
import functools

import numpy as np

import jax
import jax.numpy as jnp
from jax import lax
from jax.experimental import pallas as pl
from jax.experimental.pallas import tpu as pltpu
from jax.experimental.pallas import tpu_sc as plsc

N = 10000
E = 320000
D = 128

NC = 2
NS = 16
NW = NC * NS
NP = 10240
EP = NW * NP
EPAD = EP - E
C = 128
G = 8
TOTCH = EP // C
NCH0 = 112
NCH1 = 48
NG0 = NCH0 // G
NG1 = NCH1 // G
RPT = NP // NS
ZR = 8
DR = NP // D
L = 16

_M = np.empty((D,), np.int64)
for _j in range(D // 32):
    for _r in range(32):
        _M[32 * _j + _r] = 32 * _j + (2 * _r if _r < 16 else 2 * (_r - 16) + 1)
_MINV = np.argsort(_M)
_PM = np.zeros((D, D), np.float32)
for _b in range(D):
    _PM[_MINV[_b], _b] = 1.0


def _agg_body(compute_deg, *refs):
    if compute_deg:
        (feat, srcI, dstI, out, degout,
         srcg0, dstg0, srcg1, dstg1, rowsA, rowsB, rowsF, zbuf, degl, acc,
         semA, semB, semI0, semI1) = refs
    else:
        (feat, srcI, dstI, out,
         srcg0, dstg0, srcg1, dstg1, rowsA, rowsB, rowsF, zbuf, acc,
         semA, semB, semI0, semI1) = refs
    c = lax.axis_index("c")
    s = lax.axis_index("s")

    zeros = jnp.zeros((L,), jnp.float32)
    ones = jnp.ones((L,), jnp.float32)

    if True:
        def zrow(i, _):
            def zcol(j, _):
                zbuf[i, pl.ds(j * L, L)] = zeros
                return 0
            return lax.fori_loop(0, D // L, zcol, 0)

        lax.fori_loop(0, ZR, zrow, 0)

        def zslice(r, _):
            pltpu.sync_copy(zbuf, acc.at[pl.ds(s * RPT + r * ZR, ZR)])
            return 0

        lax.fori_loop(0, RPT // ZR, zslice, 0)

        if compute_deg:
            def zdeg(i, _):
                def zdcol(j, _):
                    degl[i, pl.ds(j * L, L)] = zeros
                    return 0
                return lax.fori_loop(0, D // L, zdcol, 0)
            lax.fori_loop(0, DR, zdeg, 0)

    plsc.subcore_barrier()

    def fire_gather(sg, row, rbuf, sem):
        pltpu.async_copy(feat.at[sg.at[row]], rbuf, sem)

    def wait_gather(sg, rbuf, sem):
        pltpu.make_async_copy(feat.at[sg.at[0]], rbuf, sem).wait()

    def do_scatter(rbuf, dg, row):
        mask = jnp.full((L,), -65536, jnp.int32)

        @plsc.parallel_loop(0, C, unroll=4)
        def conv_row(r):
            for j in range(D // 32):
                b = rbuf[r, pl.ds(j * 32, 32)]
                w = plsc.bitcast(b, jnp.int32)
                lo = lax.shift_left(w, 16)
                hi = lax.bitwise_and(w, mask)
                rowsF[r, pl.ds(j * 32, L)] = plsc.bitcast(lo, jnp.float32)
                rowsF[r, pl.ds(j * 32 + L, L)] = plsc.bitcast(hi, jnp.float32)
        pltpu.sync_copy(rowsF, acc.at[dg.at[row]], add=True)
        if compute_deg:
            def dv(i, _):
                d = dg[row, pl.ds(i * L, L)]
                plsc.addupdate_scatter(
                    degl,
                    [lax.shift_right_logical(d, 7),
                     lax.bitwise_and(d, 127)],
                    ones)
                return 0
            lax.fori_loop(0, C // L, dv, 0)

    def run_pipeline(cb, ng):
        def fire_stage(g, sg, dg, sem):
            pltpu.async_copy(srcI.at[pl.ds(cb + g * G, G)], sg, sem)
            pltpu.async_copy(dstI.at[pl.ds(cb + g * G, G)], dg, sem)

        def wait_stage(sg, dg, sem):
            pltpu.make_async_copy(srcI.at[pl.ds(cb, G)], sg, sem).wait()
            pltpu.make_async_copy(dstI.at[pl.ds(cb, G)], dg, sem).wait()

        def block(sg, dg, nsg, ndg, semNxt, fire_next, restage_g,
                  restage_sem):
            def pair(kk, _):
                fire_gather(sg, 2 * kk + 1, rowsB, semB)
                wait_gather(sg, rowsA, semA)
                do_scatter(rowsA, dg, 2 * kk)
                fire_gather(sg, 2 * kk + 2, rowsA, semA)
                wait_gather(sg, rowsB, semB)
                do_scatter(rowsB, dg, 2 * kk + 1)
                return 0

            lax.fori_loop(0, G // 2 - 1, pair, 0)
            fire_gather(sg, G - 1, rowsB, semB)
            wait_gather(sg, rowsA, semA)
            do_scatter(rowsA, dg, G - 2)

            @pl.when(fire_next)
            def _():
                wait_stage(nsg, ndg, semNxt)
                fire_gather(nsg, 0, rowsA, semA)

            wait_gather(sg, rowsB, semB)
            do_scatter(rowsB, dg, G - 1)

            @pl.when(restage_g < ng)
            def _():
                fire_stage(restage_g, sg, dg, restage_sem)

        pltpu.sync_copy(srcI.at[pl.ds(cb, G)], srcg0)
        pltpu.sync_copy(dstI.at[pl.ds(cb, G)], dstg0)
        fire_stage(1, srcg1, dstg1, semI1)
        fire_gather(srcg0, 0, rowsA, semA)

        ng2 = ng // 2

        def gloop(gp, _):
            block(srcg0, dstg0, srcg1, dstg1, semI1,
                  fire_next=gp < ng2, restage_g=2 * gp + 2,
                  restage_sem=semI0)
            block(srcg1, dstg1, srcg0, dstg0, semI0,
                  fire_next=gp < ng2 - 1, restage_g=2 * gp + 3,
                  restage_sem=semI1)
            return 0

        lax.fori_loop(0, ng2, gloop, 0)

    @pl.when(c == 0)
    def _():
        run_pipeline(s * NCH0, NG0)

    @pl.when(c == 1)
    def _():
        run_pipeline(NS * NCH0 + s * NCH1, NG1)

    plsc.subcore_barrier()

    pltpu.sync_copy(acc.at[pl.ds(s * RPT, RPT)],
                    out.at[c, pl.ds(s * RPT, RPT)])
    if compute_deg:
        pltpu.sync_copy(degl, degout.at[s * NC + c])


@functools.lru_cache(maxsize=None)
def _make_agg(compute_deg):
    mesh = plsc.VectorSubcoreMesh(core_axis_name="c", subcore_axis_name="s",
                                  num_cores=NC, num_subcores=NS)
    out_type = [jax.ShapeDtypeStruct((NC, NP, D), jnp.float32)]
    scratch = [
        pltpu.VMEM((G, C), jnp.int32),
        pltpu.VMEM((G, C), jnp.int32),
        pltpu.VMEM((G, C), jnp.int32),
        pltpu.VMEM((G, C), jnp.int32),
        pltpu.VMEM((C, D), jnp.bfloat16),
        pltpu.VMEM((C, D), jnp.bfloat16),
        pltpu.VMEM((C, D), jnp.float32),
        pltpu.VMEM((ZR, D), jnp.float32),
    ]
    if compute_deg:
        out_type.append(jax.ShapeDtypeStruct((NW, DR, D), jnp.float32))
        scratch.append(pltpu.VMEM((DR, D), jnp.float32))
    scratch.append(pltpu.VMEM_SHARED((NP, D), jnp.float32))
    scratch.extend([pltpu.SemaphoreType.DMA] * 4)
    return pl.kernel(
        functools.partial(_agg_body, compute_deg),
        out_type=tuple(out_type),
        mesh=mesh,
        scratch_types=tuple(scratch),
        compiler_params=pltpu.CompilerParams(needs_layout_passes=False,
                                             use_tc_tiling_on_sc=False),
    )


def _tc1_body(p, inv, x, wl, bl, wr, pm, out, outp):
    agg = (p[0] + p[1]) * inv[...]
    h = (jnp.dot(agg, wl[...], preferred_element_type=jnp.float32)
         + bl[...]
         + jnp.dot(x[...], wr[...], preferred_element_type=jnp.float32))
    h = jnp.maximum(h, 0.0)
    out[...] = h
    outp[...] = jnp.dot(h, pm[...],
                        preferred_element_type=jnp.float32).astype(jnp.bfloat16)


def _tc2_body(q, inv, e1, wl2, bl2, wr2, wfc1, bfc1, g1, be1,
              wfc2, bfc2, g2, be2, wfc3, bfc3, emb2, out):
    agg = (q[0] + q[1]) * inv[...]
    h = (jnp.dot(agg, wl2[...], preferred_element_type=jnp.float32)
         + bl2[...]
         + jnp.dot(e1[...], wr2[...], preferred_element_type=jnp.float32))
    h = jnp.maximum(h, 0.0)
    emb2[...] = h

    y = jnp.dot(h, wfc1[...], preferred_element_type=jnp.float32) + bfc1[...]
    mu = jnp.mean(y, axis=0, keepdims=True)
    var = jnp.mean((y - mu) ** 2, axis=0, keepdims=True)
    y = g1[...] * (y - mu) / jnp.sqrt(var + 1e-5) + be1[...]
    y = jnp.maximum(y, 0.0)

    y = jnp.dot(y, wfc2[...], preferred_element_type=jnp.float32) + bfc2[...]
    mu = jnp.mean(y, axis=0, keepdims=True)
    var = jnp.mean((y - mu) ** 2, axis=0, keepdims=True)
    y = g2[...] * (y - mu) / jnp.sqrt(var + 1e-5) + be2[...]
    y = jnp.maximum(y, 0.0)

    out[...] = jnp.dot(y, wfc3[...], preferred_element_type=jnp.float32) + bfc3[...]


def kernel(x, edge_index, W_l1, b_l1, W_r1, W_l2, b_l2, W_r2,
           W_fc1, b_fc1, g_bn1, be_bn1, W_fc2, b_fc2, g_bn2, be_bn2,
           W_fc3, b_fc3):
    ei = edge_index.astype(jnp.int32)
    srcI = jnp.concatenate(
        [ei[0], jnp.zeros((EPAD,), jnp.int32)]).reshape(TOTCH, C)
    dstI = jnp.concatenate(
        [ei[1], jnp.full((EPAD,), N, jnp.int32)]).reshape(TOTCH, C)

    xp_bf = x[:, _MINV].astype(jnp.bfloat16)
    p1, degp = _make_agg(True)(xp_bf, srcI, dstI)

    deg = degp.sum(axis=0).reshape(NP)[:N]
    inv = (1.0 / jnp.maximum(deg, 1.0)).reshape(N, 1)

    def _full(a):
        return pl.BlockSpec(a.shape, lambda i: (0,) * a.ndim)

    _pspec = pl.BlockSpec((NC, N, D), lambda i: (0, 0, 0))

    tc1_in = [p1, inv, x, W_l1, b_l1.reshape(1, -1), W_r1, jnp.asarray(_PM)]
    emb1, emb1p_bf = pl.pallas_call(
        _tc1_body,
        out_shape=(jax.ShapeDtypeStruct((N, D), jnp.float32),
                   jax.ShapeDtypeStruct((N, D), jnp.bfloat16)),
        grid=(1,),
        in_specs=[_pspec] + [_full(a) for a in tc1_in[1:]],
        out_specs=(pl.BlockSpec((N, D), lambda i: (0, 0)),
                   pl.BlockSpec((N, D), lambda i: (0, 0))),
    )(*tc1_in)

    (q,) = _make_agg(False)(emb1p_bf, srcI, dstI)

    tc2_in = [q, inv, emb1, W_l2, b_l2.reshape(1, -1), W_r2,
              W_fc1, b_fc1.reshape(1, -1), g_bn1.reshape(1, -1),
              be_bn1.reshape(1, -1), W_fc2, b_fc2.reshape(1, -1),
              g_bn2.reshape(1, -1), be_bn2.reshape(1, -1),
              W_fc3, b_fc3.reshape(1, -1)]
    emb2, out = pl.pallas_call(
        _tc2_body,
        out_shape=(
            jax.ShapeDtypeStruct((N, D), jnp.float32),
            jax.ShapeDtypeStruct((N, 1), jnp.float32),
        ),
        grid=(1,),
        in_specs=[_pspec] + [_full(a) for a in tc2_in[1:]],
        out_specs=(pl.BlockSpec((N, D), lambda i: (0, 0)),
                   pl.BlockSpec((N, 1), lambda i: (0, 0))),
    )(*tc2_in)

    return (out.reshape(N), emb1, emb2)
